# Initial kernel scaffold; baseline (speedup 1.0000x reference)
import jax
import jax.numpy as jnp
from jax.experimental import pallas as pl


def kernel(xyz, xy, xz, yz, W_mlp):
    raise NotImplementedError("write your pallas kernel here")


# speedup vs baseline: 25.4524x; 25.4524x over previous
import functools

import jax
import jax.numpy as jnp
from jax import lax
from jax.experimental import pallas as pl
from jax.experimental.pallas import tpu as pltpu
from jax.experimental.pallas import tpu_sc as plsc

RES = 256
RANK = 16
OUT_D = 32
NC, NS, LANES = 2, 16, 16
NW = NC * NS
CP = 128
NB = CP // LANES


def _lpf_body(p_ref, o_ref):
    p = p_ref[...]
    zy = jnp.zeros((p.shape[0], 1, RES), jnp.float32)
    sy = (p + jnp.concatenate([p[:, 1:, :], zy], axis=1)
          + jnp.concatenate([zy, p[:, :-1, :]], axis=1))
    zx = jnp.zeros((p.shape[0], RES, 1), jnp.float32)
    sx = (sy + jnp.concatenate([sy[:, :, 1:], zx], axis=2)
          + jnp.concatenate([zx, sy[:, :, :-1]], axis=2))
    o_ref[...] = sx * (1.0 / 9.0)


def _mm_body(f_ref, w_ref, o_ref):
    o_ref[...] = lax.dot_general(
        f_ref[...], w_ref[...], (((1,), (1,)), ((), ())),
        preferred_element_type=jnp.float32)


def _sanitize(t):
    d = t - t
    t = jnp.where(d != d, jnp.float32(0.5), t)
    return jnp.minimum(jnp.maximum(t, jnp.float32(0.0)), jnp.float32(1.0))


def _bilinear_uv(u, v, plane_base):
    eps = jnp.float32(2.0 / RES)
    one = jnp.float32(1.0)
    un = jnp.minimum(jnp.maximum(u * 2.0 - 1.0, -one + eps), one - eps)
    vn = jnp.minimum(jnp.maximum(v * 2.0 - 1.0, -one + eps), one - eps)
    ix = ((un + 1.0) * RES - 1.0) * 0.5
    iy = ((vn + 1.0) * RES - 1.0) * 0.5
    x0 = ix.astype(jnp.int32)
    y0 = iy.astype(jnp.int32)
    wx1 = ix - x0.astype(jnp.float32)
    wy1 = iy - y0.astype(jnp.float32)
    wx0 = 1.0 - wx1
    wy0 = 1.0 - wy1
    f0 = y0 * RES + x0 + plane_base
    return f0, wy0 * wx0, wy0 * wx1, wy1 * wx0, wy1 * wx1


def _make_sc_call(cpw, n_pad):
    mesh = plsc.VectorSubcoreMesh(core_axis_name="c", subcore_axis_name="s")

    @functools.partial(
        pl.kernel,
        out_type=jax.ShapeDtypeStruct((n_pad, 3 * RANK), jnp.float32),
        mesh=mesh,
        scratch_types=[
            pltpu.VMEM((3, CP), jnp.float32),
            pltpu.VMEM((6, CP), jnp.int32),
            pltpu.VMEM((12, CP), jnp.float32),
            pltpu.VMEM((6, CP, 2 * RANK), jnp.float32),
            pltpu.VMEM((CP, 3 * RANK), jnp.float32),
            pltpu.SemaphoreType.DMA,
        ],
        compiler_params=pltpu.CompilerParams(
            needs_layout_passes=False, use_tc_tiling_on_sc=False),
    )
    def sc_call(x_h, y_h, z_h, t2, feat, cbuf, ibuf, wbuf, gbuf, fbuf, sem):
        wid = lax.axis_index("s") * NC + lax.axis_index("c")
        lane = lax.iota(jnp.int32, 16)

        def chunk(ci, carry):
            base = (wid * cpw + ci) * CP
            for r, src in enumerate((x_h, y_h, z_h)):
                pltpu.sync_copy(src.at[pl.ds(base, CP)], cbuf.at[r])

            def pass1(b, c2):
                off = b * LANES
                xv = _sanitize(cbuf[0, pl.ds(off, LANES)])
                yv = _sanitize(cbuf[1, pl.ds(off, LANES)])
                zv = _sanitize(cbuf[2, pl.ds(off, LANES)])
                for p, (u, v) in enumerate(((xv, yv), (xv, zv), (yv, zv))):
                    f0, wa, wb, wc, wd = _bilinear_uv(u, v, p * RES * RES)
                    ibuf[2 * p, pl.ds(off, LANES)] = f0
                    ibuf[2 * p + 1, pl.ds(off, LANES)] = f0 + RES
                    wbuf[4 * p + 0, pl.ds(off, LANES)] = wa
                    wbuf[4 * p + 1, pl.ds(off, LANES)] = wb
                    wbuf[4 * p + 2, pl.ds(off, LANES)] = wc
                    wbuf[4 * p + 3, pl.ds(off, LANES)] = wd
                return c2

            lax.fori_loop(0, NB, pass1, 0)

            copies = [pltpu.async_copy(t2.at[ibuf.at[k]], gbuf.at[k], sem)
                      for k in range(6)]
            for c in copies:
                c.wait()

            def pass2(b, c2):
                off = b * LANES
                rows = off + lane
                w = [wbuf[j, pl.ds(off, LANES)] for j in range(12)]
                for p in range(3):
                    wa, wb, wc, wd = w[4 * p:4 * p + 4]
                    k0 = jnp.full((16,), 2 * p, jnp.int32)
                    k1 = jnp.full((16,), 2 * p + 1, jnp.int32)
                    for f in range(RANK):
                        fc0 = jnp.full((16,), f, jnp.int32)
                        fc1 = jnp.full((16,), f + RANK, jnp.int32)
                        acc = wa * plsc.load_gather(gbuf, [k0, rows, fc0])
                        acc = acc + wb * plsc.load_gather(gbuf, [k0, rows, fc1])
                        acc = acc + wc * plsc.load_gather(gbuf, [k1, rows, fc0])
                        acc = acc + wd * plsc.load_gather(gbuf, [k1, rows, fc1])
                        col = jnp.full((16,), p * RANK + f, jnp.int32)
                        plsc.store_scatter(fbuf, [rows, col], acc)
                return c2

            lax.fori_loop(0, NB, pass2, 0)
            pltpu.sync_copy(fbuf, feat.at[pl.ds(base, CP)])
            return carry

        lax.fori_loop(0, cpw, chunk, 0)

    return sc_call


def kernel(xyz, xy, xz, yz, W_mlp):
    n = xyz.shape[0]
    chunks = -(-n // CP)
    cpw = -(-chunks // NW)
    n_pad = cpw * NW * CP

    planes = jnp.concatenate([xy, xz, yz], axis=0)
    lpf = pl.pallas_call(
        _lpf_body,
        grid=(planes.shape[0] // 8,),
        in_specs=[pl.BlockSpec((8, RES, RES), lambda i: (i, 0, 0))],
        out_specs=pl.BlockSpec((8, RES, RES), lambda i: (i, 0, 0)),
        out_shape=jax.ShapeDtypeStruct(planes.shape, jnp.float32),
    )(planes)

    p_t = lpf.reshape(3, RANK, RES, RES).transpose(0, 2, 3, 1)
    p_sh = jnp.concatenate([p_t[:, :, 1:, :], p_t[:, :, -1:, :]], axis=2)
    t2 = jnp.concatenate([p_t, p_sh], axis=-1).reshape(3 * RES * RES, 2 * RANK)

    pads = ((0, n_pad - n),)
    xs = jnp.pad(xyz[:, 0], pads, constant_values=0.5)
    ys = jnp.pad(xyz[:, 1], pads, constant_values=0.5)
    zs = jnp.pad(xyz[:, 2], pads, constant_values=0.5)
    feat = _make_sc_call(cpw, n_pad)(xs, ys, zs, t2)

    out = pl.pallas_call(
        _mm_body,
        grid=(n_pad // 4096,),
        in_specs=[pl.BlockSpec((4096, 3 * RANK), lambda i: (i, 0)),
                  pl.BlockSpec((OUT_D, 3 * RANK), lambda i: (0, 0))],
        out_specs=pl.BlockSpec((4096, OUT_D), lambda i: (i, 0)),
        out_shape=jax.ShapeDtypeStruct((n_pad, OUT_D), jnp.float32),
    )(feat, W_mlp)
    return out[:n]


# speedup vs baseline: 27.4410x; 1.0781x over previous
import functools

import jax
import jax.numpy as jnp
from jax import lax
from jax.experimental import pallas as pl
from jax.experimental.pallas import tpu as pltpu
from jax.experimental.pallas import tpu_sc as plsc

RES = 256
RANK = 16
OUT_D = 32
NC, NS, LANES = 2, 16, 16
NW = NC * NS
CP = 128
NB = CP // LANES


def _lpf_body(p_ref, o_ref):
    p = p_ref[...]
    zy = jnp.zeros((p.shape[0], 1, RES), jnp.float32)
    sy = (p + jnp.concatenate([p[:, 1:, :], zy], axis=1)
          + jnp.concatenate([zy, p[:, :-1, :]], axis=1))
    zx = jnp.zeros((p.shape[0], RES, 1), jnp.float32)
    sx = (sy + jnp.concatenate([sy[:, :, 1:], zx], axis=2)
          + jnp.concatenate([zx, sy[:, :, :-1]], axis=2))
    o_ref[...] = sx * (1.0 / 9.0)


def _mm_body(f_ref, w_ref, o_ref):
    o_ref[...] = lax.dot_general(
        f_ref[...], w_ref[...], (((1,), (1,)), ((), ())),
        preferred_element_type=jnp.float32)


def _sanitize(t):
    d = t - t
    t = jnp.where(d != d, jnp.float32(0.5), t)
    return jnp.minimum(jnp.maximum(t, jnp.float32(0.0)), jnp.float32(1.0))


def _bilinear_uv(u, v, plane_base):
    eps = jnp.float32(2.0 / RES)
    one = jnp.float32(1.0)
    un = jnp.minimum(jnp.maximum(u * 2.0 - 1.0, -one + eps), one - eps)
    vn = jnp.minimum(jnp.maximum(v * 2.0 - 1.0, -one + eps), one - eps)
    ix = ((un + 1.0) * RES - 1.0) * 0.5
    iy = ((vn + 1.0) * RES - 1.0) * 0.5
    x0 = ix.astype(jnp.int32)
    y0 = iy.astype(jnp.int32)
    wx1 = ix - x0.astype(jnp.float32)
    wy1 = iy - y0.astype(jnp.float32)
    wx0 = 1.0 - wx1
    wy0 = 1.0 - wy1
    f0 = y0 * RES + x0 + plane_base
    return f0, wy0 * wx0, wy0 * wx1, wy1 * wx0, wy1 * wx1


def _make_sc_call(cpw, n_pad):
    assert cpw % 2 == 0
    mesh = plsc.VectorSubcoreMesh(core_axis_name="c", subcore_axis_name="s")

    @functools.partial(
        pl.kernel,
        out_type=jax.ShapeDtypeStruct((n_pad, 3 * RANK), jnp.float32),
        mesh=mesh,
        scratch_types=[
            pltpu.VMEM((3, 2 * CP), jnp.float32),
            pltpu.VMEM((2, 6, CP), jnp.int32),
            pltpu.VMEM((2, 12, CP), jnp.float32),
            pltpu.VMEM((2, 6, CP, 2 * RANK), jnp.float32),
            pltpu.VMEM((2, CP, 3 * RANK), jnp.float32),
            pltpu.SemaphoreType.DMA,
            pltpu.SemaphoreType.DMA,
            pltpu.SemaphoreType.DMA,
            pltpu.SemaphoreType.DMA,
        ],
        compiler_params=pltpu.CompilerParams(
            needs_layout_passes=False, use_tc_tiling_on_sc=False),
    )
    def sc_call(x_h, y_h, z_h, t2, feat, cbuf, ibuf, wbuf, gbuf, fbuf,
                sg0, sg1, sf0, sf1):
        wid = lax.axis_index("s") * NC + lax.axis_index("c")
        lane = lax.iota(jnp.int32, 16)
        wbase = wid * cpw * CP
        semg = (sg0, sg1)
        semf = (sf0, sf1)

        def stage_coords(first_chunk):
            off = wbase + first_chunk * CP
            for r, src in enumerate((x_h, y_h, z_h)):
                pltpu.sync_copy(src.at[pl.ds(off, 2 * CP)], cbuf.at[r])

        def pass1(s, half):
            def body(b, c2):
                off = half * CP + b * LANES
                doff = b * LANES
                xv = _sanitize(cbuf[0, pl.ds(off, LANES)])
                yv = _sanitize(cbuf[1, pl.ds(off, LANES)])
                zv = _sanitize(cbuf[2, pl.ds(off, LANES)])
                for p, (u, v) in enumerate(((xv, yv), (xv, zv), (yv, zv))):
                    f0, wa, wb, wc, wd = _bilinear_uv(u, v, p * RES * RES)
                    ibuf[s, 2 * p, pl.ds(doff, LANES)] = f0
                    ibuf[s, 2 * p + 1, pl.ds(doff, LANES)] = f0 + RES
                    wbuf[s, 4 * p + 0, pl.ds(doff, LANES)] = wa
                    wbuf[s, 4 * p + 1, pl.ds(doff, LANES)] = wb
                    wbuf[s, 4 * p + 2, pl.ds(doff, LANES)] = wc
                    wbuf[s, 4 * p + 3, pl.ds(doff, LANES)] = wd
                return c2

            lax.fori_loop(0, NB, body, 0)

        def fire_gathers(s):
            return [pltpu.async_copy(t2.at[ibuf.at[s, k]], gbuf.at[s, k],
                                     semg[s])
                    for k in range(6)]

        def pass2(s):
            def body(b, c2):
                off = b * LANES
                rows = off + lane
                w = [wbuf[s, j, pl.ds(off, LANES)] for j in range(12)]
                sfull = jnp.full((16,), s, jnp.int32)
                for p in range(3):
                    wa, wb, wc, wd = w[4 * p:4 * p + 4]
                    k0 = jnp.full((16,), 2 * p, jnp.int32)
                    k1 = jnp.full((16,), 2 * p + 1, jnp.int32)
                    for f in range(RANK):
                        fc0 = jnp.full((16,), f, jnp.int32)
                        fc1 = jnp.full((16,), f + RANK, jnp.int32)
                        acc = wa * plsc.load_gather(gbuf, [sfull, k0, rows, fc0])
                        acc = acc + wb * plsc.load_gather(gbuf, [sfull, k0, rows, fc1])
                        acc = acc + wc * plsc.load_gather(gbuf, [sfull, k1, rows, fc0])
                        acc = acc + wd * plsc.load_gather(gbuf, [sfull, k1, rows, fc1])
                        col = jnp.full((16,), p * RANK + f, jnp.int32)
                        plsc.store_scatter(fbuf.at[s], [rows, col], acc)
                return c2

            lax.fori_loop(0, NB, body, 0)

        def drain_featw(s):
            pltpu.make_async_copy(feat.at[pl.ds(0, CP)], fbuf.at[s],
                                  semf[s]).wait()

        def fire_featw(s, ci):
            pltpu.async_copy(fbuf.at[s], feat.at[pl.ds(wbase + ci * CP, CP)],
                             semf[s])

        stage_coords(0)

        def pair(k, carry):
            c0 = 2 * k
            pass1(0, 0)
            g0 = fire_gathers(0)
            pass1(1, 1)
            g1 = fire_gathers(1)

            @pl.when(k < cpw // 2 - 1)
            def _():
                stage_coords(c0 + 2)

            for c in g0:
                c.wait()

            @pl.when(k >= 1)
            def _():
                drain_featw(0)

            pass2(0)
            fire_featw(0, c0)
            for c in g1:
                c.wait()

            @pl.when(k >= 1)
            def _():
                drain_featw(1)

            pass2(1)
            fire_featw(1, c0 + 1)
            return carry

        lax.fori_loop(0, cpw // 2, pair, 0)
        drain_featw(0)
        drain_featw(1)

    return sc_call


def kernel(xyz, xy, xz, yz, W_mlp):
    n = xyz.shape[0]
    chunks = -(-n // CP)
    cpw = -(-chunks // NW)
    cpw += cpw % 2
    n_pad = cpw * NW * CP

    planes = jnp.concatenate([xy, xz, yz], axis=0)
    lpf = pl.pallas_call(
        _lpf_body,
        grid=(planes.shape[0] // 8,),
        in_specs=[pl.BlockSpec((8, RES, RES), lambda i: (i, 0, 0))],
        out_specs=pl.BlockSpec((8, RES, RES), lambda i: (i, 0, 0)),
        out_shape=jax.ShapeDtypeStruct(planes.shape, jnp.float32),
    )(planes)

    p_t = lpf.reshape(3, RANK, RES, RES).transpose(0, 2, 3, 1)
    p_sh = jnp.concatenate([p_t[:, :, 1:, :], p_t[:, :, -1:, :]], axis=2)
    t2 = jnp.concatenate([p_t, p_sh], axis=-1).reshape(3 * RES * RES, 2 * RANK)

    pads = ((0, n_pad - n),)
    xs = jnp.pad(xyz[:, 0], pads, constant_values=0.5)
    ys = jnp.pad(xyz[:, 1], pads, constant_values=0.5)
    zs = jnp.pad(xyz[:, 2], pads, constant_values=0.5)
    feat = _make_sc_call(cpw, n_pad)(xs, ys, zs, t2)

    out = pl.pallas_call(
        _mm_body,
        grid=(n_pad // 4096,),
        in_specs=[pl.BlockSpec((4096, 3 * RANK), lambda i: (i, 0)),
                  pl.BlockSpec((OUT_D, 3 * RANK), lambda i: (0, 0))],
        out_specs=pl.BlockSpec((4096, OUT_D), lambda i: (i, 0)),
        out_shape=jax.ShapeDtypeStruct((n_pad, OUT_D), jnp.float32),
    )(feat, W_mlp)
    return out[:n]


# speedup vs baseline: 51.0247x; 1.8594x over previous
import functools

import jax
import jax.numpy as jnp
from jax import lax
from jax.experimental import pallas as pl
from jax.experimental.pallas import tpu as pltpu
from jax.experimental.pallas import tpu_sc as plsc

RES = 256
RANK = 16
OUT_D = 32
NC, NS, LANES = 2, 16, 16
NW = NC * NS
CP = 128
NB = CP // LANES


def _lpf_body(p_ref, o_ref):
    p = p_ref[...]
    zy = jnp.zeros((p.shape[0], 1, RES), jnp.float32)
    sy = (p + jnp.concatenate([p[:, 1:, :], zy], axis=1)
          + jnp.concatenate([zy, p[:, :-1, :]], axis=1))
    zx = jnp.zeros((p.shape[0], RES, 1), jnp.float32)
    sx = (sy + jnp.concatenate([sy[:, :, 1:], zx], axis=2)
          + jnp.concatenate([zx, sy[:, :, :-1]], axis=2))
    o_ref[...] = sx * (1.0 / 9.0)


def _mm_body(f_ref, w_ref, o_ref):
    o_ref[...] = lax.dot_general(
        f_ref[...], w_ref[...], (((1,), (1,)), ((), ())),
        preferred_element_type=jnp.float32)


def _sanitize(t):
    d = t - t
    t = jnp.where(d != d, jnp.float32(0.5), t)
    return jnp.minimum(jnp.maximum(t, jnp.float32(0.0)), jnp.float32(1.0))


def _bilinear_uv(u, v, plane_base):
    eps = jnp.float32(2.0 / RES)
    one = jnp.float32(1.0)
    un = jnp.minimum(jnp.maximum(u * 2.0 - 1.0, -one + eps), one - eps)
    vn = jnp.minimum(jnp.maximum(v * 2.0 - 1.0, -one + eps), one - eps)
    ix = ((un + 1.0) * RES - 1.0) * 0.5
    iy = ((vn + 1.0) * RES - 1.0) * 0.5
    x0 = ix.astype(jnp.int32)
    y0 = iy.astype(jnp.int32)
    wx1 = ix - x0.astype(jnp.float32)
    wy1 = iy - y0.astype(jnp.float32)
    wx0 = 1.0 - wx1
    wy0 = 1.0 - wy1
    f0 = y0 * RES + x0 + plane_base
    return f0, wy0 * wx0, wy0 * wx1, wy1 * wx0, wy1 * wx1


def _make_sc_call(cpw, n_pad):
    assert cpw % 2 == 0
    mesh = plsc.VectorSubcoreMesh(core_axis_name="c", subcore_axis_name="s")

    @functools.partial(
        pl.kernel,
        out_type=jax.ShapeDtypeStruct((n_pad, 3 * RANK), jnp.float32),
        mesh=mesh,
        scratch_types=[
            pltpu.VMEM((3, 2 * CP), jnp.float32),
            pltpu.VMEM((2, 3, CP), jnp.int32),
            pltpu.VMEM((2, 12, CP), jnp.float32),
            pltpu.VMEM((2, 3, CP, 4 * RANK), jnp.float32),
            pltpu.VMEM((2, CP, 3 * RANK), jnp.float32),
            pltpu.SemaphoreType.DMA,
            pltpu.SemaphoreType.DMA,
            pltpu.SemaphoreType.DMA,
            pltpu.SemaphoreType.DMA,
        ],
        compiler_params=pltpu.CompilerParams(
            needs_layout_passes=False, use_tc_tiling_on_sc=False),
    )
    def sc_call(x_h, y_h, z_h, t4, feat, cbuf, ibuf, wbuf, gbuf, fbuf,
                sg0, sg1, sf0, sf1):
        wid = lax.axis_index("s") * NC + lax.axis_index("c")
        lane = lax.iota(jnp.int32, 16)
        wbase = wid * cpw * CP
        semg = (sg0, sg1)
        semf = (sf0, sf1)

        def stage_coords(first_chunk):
            off = wbase + first_chunk * CP
            for r, src in enumerate((x_h, y_h, z_h)):
                pltpu.sync_copy(src.at[pl.ds(off, 2 * CP)], cbuf.at[r])

        def pass1(s, half):
            def body(b):
                off = half * CP + b * LANES
                doff = b * LANES
                xv = _sanitize(cbuf[0, pl.ds(off, LANES)])
                yv = _sanitize(cbuf[1, pl.ds(off, LANES)])
                zv = _sanitize(cbuf[2, pl.ds(off, LANES)])
                for p, (u, v) in enumerate(((xv, yv), (xv, zv), (yv, zv))):
                    f0, wa, wb, wc, wd = _bilinear_uv(u, v, p * RES * RES)
                    ibuf[s, p, pl.ds(doff, LANES)] = f0
                    wbuf[s, 4 * p + 0, pl.ds(doff, LANES)] = wa
                    wbuf[s, 4 * p + 1, pl.ds(doff, LANES)] = wb
                    wbuf[s, 4 * p + 2, pl.ds(doff, LANES)] = wc
                    wbuf[s, 4 * p + 3, pl.ds(doff, LANES)] = wd

            plsc.parallel_loop(0, NB, 1)(body)

        def fire_gathers(s):
            return [pltpu.async_copy(t4.at[ibuf.at[s, k]], gbuf.at[s, k],
                                     semg[s])
                    for k in range(3)]

        def pass2(s):
            def body(b):
                off = b * LANES
                w = [wbuf[s, j, pl.ds(off, LANES)] for j in range(12)]
                for j in range(LANES):
                    n = off + j
                    for p in range(3):
                        wa = w[4 * p + 0][j]
                        wb = w[4 * p + 1][j]
                        wc = w[4 * p + 2][j]
                        wd = w[4 * p + 3][j]
                        r0a = gbuf[s, p, n, pl.ds(0, RANK)]
                        r0b = gbuf[s, p, n, pl.ds(RANK, RANK)]
                        r1a = gbuf[s, p, n, pl.ds(2 * RANK, RANK)]
                        r1b = gbuf[s, p, n, pl.ds(3 * RANK, RANK)]
                        acc = (wa * r0a + wb * r0b) + (wc * r1a + wd * r1b)
                        fbuf[s, n, pl.ds(p * RANK, RANK)] = acc

            plsc.parallel_loop(0, NB, 1)(body)

        def drain_featw(s):
            pltpu.make_async_copy(feat.at[pl.ds(0, CP)], fbuf.at[s],
                                  semf[s]).wait()

        def fire_featw(s, ci):
            pltpu.async_copy(fbuf.at[s], feat.at[pl.ds(wbase + ci * CP, CP)],
                             semf[s])

        stage_coords(0)

        def pair(k, carry):
            c0 = 2 * k
            pass1(0, 0)
            g0 = fire_gathers(0)
            pass1(1, 1)
            g1 = fire_gathers(1)

            @pl.when(k < cpw // 2 - 1)
            def _():
                stage_coords(c0 + 2)

            for c in g0:
                c.wait()

            @pl.when(k >= 1)
            def _():
                drain_featw(0)

            pass2(0)
            fire_featw(0, c0)
            for c in g1:
                c.wait()

            @pl.when(k >= 1)
            def _():
                drain_featw(1)

            pass2(1)
            fire_featw(1, c0 + 1)
            return carry

        lax.fori_loop(0, cpw // 2, pair, 0)
        drain_featw(0)
        drain_featw(1)

    return sc_call


def kernel(xyz, xy, xz, yz, W_mlp):
    n = xyz.shape[0]
    chunks = -(-n // CP)
    cpw = -(-chunks // NW)
    cpw += cpw % 2
    n_pad = cpw * NW * CP

    planes = jnp.concatenate([xy, xz, yz], axis=0)
    lpf = pl.pallas_call(
        _lpf_body,
        grid=(planes.shape[0] // 8,),
        in_specs=[pl.BlockSpec((8, RES, RES), lambda i: (i, 0, 0))],
        out_specs=pl.BlockSpec((8, RES, RES), lambda i: (i, 0, 0)),
        out_shape=jax.ShapeDtypeStruct(planes.shape, jnp.float32),
    )(planes)

    p_t = lpf.reshape(3, RANK, RES, RES).transpose(0, 2, 3, 1)
    p_sx = jnp.concatenate([p_t[:, :, 1:, :], p_t[:, :, -1:, :]], axis=2)
    q = jnp.concatenate([p_t, p_sx], axis=-1)
    q_sy = jnp.concatenate([q[:, 1:], q[:, -1:]], axis=1)
    t4 = jnp.concatenate([q, q_sy], axis=-1).reshape(3 * RES * RES, 4 * RANK)

    pads = ((0, n_pad - n),)
    xs = jnp.pad(xyz[:, 0], pads, constant_values=0.5)
    ys = jnp.pad(xyz[:, 1], pads, constant_values=0.5)
    zs = jnp.pad(xyz[:, 2], pads, constant_values=0.5)
    feat = _make_sc_call(cpw, n_pad)(xs, ys, zs, t4)

    out = pl.pallas_call(
        _mm_body,
        grid=(n_pad // 4096,),
        in_specs=[pl.BlockSpec((4096, 3 * RANK), lambda i: (i, 0)),
                  pl.BlockSpec((OUT_D, 3 * RANK), lambda i: (0, 0))],
        out_specs=pl.BlockSpec((4096, OUT_D), lambda i: (i, 0)),
        out_shape=jax.ShapeDtypeStruct((n_pad, OUT_D), jnp.float32),
    )(feat, W_mlp)
    return out[:n]


# speedup vs baseline: 55.4212x; 1.0862x over previous
import functools

import jax
import jax.numpy as jnp
from jax import lax
from jax.experimental import pallas as pl
from jax.experimental.pallas import tpu as pltpu
from jax.experimental.pallas import tpu_sc as plsc

RES = 256
RANK = 16
OUT_D = 32
NC, NS, LANES = 2, 16, 16
NW = NC * NS
CP = 128
NB = CP // LANES


def _lpf_body(p_ref, o_ref):
    p = p_ref[...]
    zy = jnp.zeros((p.shape[0], 1, RES), jnp.float32)
    sy = (p + jnp.concatenate([p[:, 1:, :], zy], axis=1)
          + jnp.concatenate([zy, p[:, :-1, :]], axis=1))
    zx = jnp.zeros((p.shape[0], RES, 1), jnp.float32)
    sx = (sy + jnp.concatenate([sy[:, :, 1:], zx], axis=2)
          + jnp.concatenate([zx, sy[:, :, :-1]], axis=2))
    o_ref[...] = sx * (1.0 / 9.0)


def _mm_body(f_ref, w_ref, o_ref):
    o_ref[...] = lax.dot_general(
        f_ref[...], w_ref[...], (((1,), (1,)), ((), ())),
        preferred_element_type=jnp.float32)


def _sanitize(t):
    d = t - t
    t = jnp.where(d != d, jnp.float32(0.5), t)
    return jnp.minimum(jnp.maximum(t, jnp.float32(0.0)), jnp.float32(1.0))


def _bilinear_uv(u, v, plane_base):
    eps = jnp.float32(2.0 / RES)
    one = jnp.float32(1.0)
    un = jnp.minimum(jnp.maximum(u * 2.0 - 1.0, -one + eps), one - eps)
    vn = jnp.minimum(jnp.maximum(v * 2.0 - 1.0, -one + eps), one - eps)
    ix = ((un + 1.0) * RES - 1.0) * 0.5
    iy = ((vn + 1.0) * RES - 1.0) * 0.5
    x0 = ix.astype(jnp.int32)
    y0 = iy.astype(jnp.int32)
    wx1 = ix - x0.astype(jnp.float32)
    wy1 = iy - y0.astype(jnp.float32)
    wx0 = 1.0 - wx1
    wy0 = 1.0 - wy1
    f0 = y0 * RES + x0 + plane_base
    return f0, wy0 * wx0, wy0 * wx1, wy1 * wx0, wy1 * wx1


def _make_sc_call(cpw, n_pad):
    assert cpw % 2 == 0
    mesh = plsc.VectorSubcoreMesh(core_axis_name="c", subcore_axis_name="s")

    @functools.partial(
        pl.kernel,
        out_type=jax.ShapeDtypeStruct((n_pad * 3 * RANK,), jnp.float32),
        mesh=mesh,
        scratch_types=[
            pltpu.VMEM((3, 2 * CP), jnp.float32),
            pltpu.VMEM((2, 3, CP), jnp.int32),
            pltpu.VMEM((2, 12, CP), jnp.float32),
            pltpu.VMEM((2, 3, CP, 4 * RANK), jnp.float32),
            pltpu.VMEM((2, CP * 3 * RANK), jnp.float32),
            pltpu.SemaphoreType.DMA,
            pltpu.SemaphoreType.DMA,
            pltpu.SemaphoreType.DMA,
            pltpu.SemaphoreType.DMA,
        ],
        compiler_params=pltpu.CompilerParams(
            needs_layout_passes=False, use_tc_tiling_on_sc=False),
    )
    def sc_call(x_h, y_h, z_h, t4, feat, cbuf, ibuf, wbuf, gbuf, fbuf,
                sg0, sg1, sf0, sf1):
        wid = lax.axis_index("s") * NC + lax.axis_index("c")
        lane = lax.iota(jnp.int32, 16)
        wbase = wid * cpw * CP
        semg = (sg0, sg1)
        semf = (sf0, sf1)

        def stage_coords(first_chunk):
            off = wbase + first_chunk * CP
            for r, src in enumerate((x_h, y_h, z_h)):
                pltpu.sync_copy(src.at[pl.ds(off, 2 * CP)], cbuf.at[r])

        def pass1(s, half):
            def body(b):
                off = half * CP + b * LANES
                doff = b * LANES
                xv = _sanitize(cbuf[0, pl.ds(off, LANES)])
                yv = _sanitize(cbuf[1, pl.ds(off, LANES)])
                zv = _sanitize(cbuf[2, pl.ds(off, LANES)])
                for p, (u, v) in enumerate(((xv, yv), (xv, zv), (yv, zv))):
                    f0, wa, wb, wc, wd = _bilinear_uv(u, v, p * RES * RES)
                    ibuf[s, p, pl.ds(doff, LANES)] = f0
                    wbuf[s, 4 * p + 0, pl.ds(doff, LANES)] = wa
                    wbuf[s, 4 * p + 1, pl.ds(doff, LANES)] = wb
                    wbuf[s, 4 * p + 2, pl.ds(doff, LANES)] = wc
                    wbuf[s, 4 * p + 3, pl.ds(doff, LANES)] = wd

            plsc.parallel_loop(0, NB, 1)(body)

        def fire_gathers(s):
            return [pltpu.async_copy(t4.at[ibuf.at[s, k]], gbuf.at[s, k],
                                     semg[s])
                    for k in range(3)]

        def pass2(s):
            def body(b):
                off = b * LANES
                w = [wbuf[s, j, pl.ds(off, LANES)] for j in range(12)]
                for j in range(LANES):
                    n = off + j
                    for p in range(3):
                        wa = w[4 * p + 0][j]
                        wb = w[4 * p + 1][j]
                        wc = w[4 * p + 2][j]
                        wd = w[4 * p + 3][j]
                        r0a = gbuf[s, p, n, pl.ds(0, RANK)]
                        r0b = gbuf[s, p, n, pl.ds(RANK, RANK)]
                        r1a = gbuf[s, p, n, pl.ds(2 * RANK, RANK)]
                        r1b = gbuf[s, p, n, pl.ds(3 * RANK, RANK)]
                        acc = (wa * r0a + wb * r0b) + (wc * r1a + wd * r1b)
                        fbuf[s, pl.ds(n * 3 * RANK + p * RANK, RANK)] = acc

            plsc.parallel_loop(0, NB, 1)(body)

        def drain_featw(s):
            pltpu.make_async_copy(feat.at[pl.ds(0, CP * 3 * RANK)],
                                  fbuf.at[s], semf[s]).wait()

        def fire_featw(s, ci):
            pltpu.async_copy(
                fbuf.at[s],
                feat.at[pl.ds((wbase + ci * CP) * 3 * RANK, CP * 3 * RANK)],
                semf[s])

        stage_coords(0)

        def pair(k, carry):
            c0 = 2 * k
            pass1(0, 0)
            g0 = fire_gathers(0)
            pass1(1, 1)
            g1 = fire_gathers(1)

            @pl.when(k < cpw // 2 - 1)
            def _():
                stage_coords(c0 + 2)

            for c in g0:
                c.wait()

            @pl.when(k >= 1)
            def _():
                drain_featw(0)

            pass2(0)
            fire_featw(0, c0)
            for c in g1:
                c.wait()

            @pl.when(k >= 1)
            def _():
                drain_featw(1)

            pass2(1)
            fire_featw(1, c0 + 1)
            return carry

        lax.fori_loop(0, cpw // 2, pair, 0)
        drain_featw(0)
        drain_featw(1)

    return sc_call


def kernel(xyz, xy, xz, yz, W_mlp):
    n = xyz.shape[0]
    chunks = -(-n // CP)
    cpw = -(-chunks // NW)
    cpw += cpw % 2
    n_pad = cpw * NW * CP

    planes = jnp.concatenate([xy, xz, yz], axis=0)
    lpf = pl.pallas_call(
        _lpf_body,
        grid=(planes.shape[0] // 8,),
        in_specs=[pl.BlockSpec((8, RES, RES), lambda i: (i, 0, 0))],
        out_specs=pl.BlockSpec((8, RES, RES), lambda i: (i, 0, 0)),
        out_shape=jax.ShapeDtypeStruct(planes.shape, jnp.float32),
    )(planes)

    p_t = lpf.reshape(3, RANK, RES, RES).transpose(0, 2, 3, 1)
    p_sx = jnp.concatenate([p_t[:, :, 1:, :], p_t[:, :, -1:, :]], axis=2)
    q = jnp.concatenate([p_t, p_sx], axis=-1)
    q_sy = jnp.concatenate([q[:, 1:], q[:, -1:]], axis=1)
    t4 = jnp.concatenate([q, q_sy], axis=-1).reshape(3 * RES * RES, 4 * RANK)

    pads = ((0, n_pad - n),)
    xs = jnp.pad(xyz[:, 0], pads, constant_values=0.5)
    ys = jnp.pad(xyz[:, 1], pads, constant_values=0.5)
    zs = jnp.pad(xyz[:, 2], pads, constant_values=0.5)
    feat = _make_sc_call(cpw, n_pad)(xs, ys, zs, t4)

    bn = 4096
    f2 = feat.reshape(n_pad, 3 * RANK)
    out = pl.pallas_call(
        _mm_body,
        grid=(-(-n // bn),),
        in_specs=[pl.BlockSpec((bn, 3 * RANK), lambda i: (i, 0)),
                  pl.BlockSpec((OUT_D, 3 * RANK), lambda i: (0, 0))],
        out_specs=pl.BlockSpec((bn, OUT_D), lambda i: (i, 0)),
        out_shape=jax.ShapeDtypeStruct((n, OUT_D), jnp.float32),
    )(f2, W_mlp)
    return out


# speedup vs baseline: 61.2827x; 1.1058x over previous
import functools

import jax
import jax.numpy as jnp
from jax import lax
from jax.experimental import pallas as pl
from jax.experimental.pallas import tpu as pltpu
from jax.experimental.pallas import tpu_sc as plsc

RES = 256
RANK = 16
OUT_D = 32
NC, NS, LANES = 2, 16, 16
NW = NC * NS
CP = 128
NB = CP // LANES


def _lpf_body(p_ref, o_ref):
    p = p_ref[...]
    zy = jnp.zeros((p.shape[0], 1, RES), jnp.float32)
    sy = (p + jnp.concatenate([p[:, 1:, :], zy], axis=1)
          + jnp.concatenate([zy, p[:, :-1, :]], axis=1))
    zx = jnp.zeros((p.shape[0], RES, 1), jnp.float32)
    sx = (sy + jnp.concatenate([sy[:, :, 1:], zx], axis=2)
          + jnp.concatenate([zx, sy[:, :, :-1]], axis=2))
    o_ref[...] = sx * (1.0 / 9.0)


def _mm_body(f_ref, w_ref, o_ref):
    f8 = f_ref[...].reshape(f_ref.shape[0] // 3, 384)
    o_ref[...] = lax.dot_general(
        f8, w_ref[...], (((1,), (0,)), ((), ())),
        preferred_element_type=jnp.float32)


def _sanitize(t):
    d = t - t
    t = jnp.where(d != d, jnp.float32(0.5), t)
    return jnp.minimum(jnp.maximum(t, jnp.float32(0.0)), jnp.float32(1.0))


def _bilinear_uv(u, v, plane_base):
    eps = jnp.float32(2.0 / RES)
    one = jnp.float32(1.0)
    un = jnp.minimum(jnp.maximum(u * 2.0 - 1.0, -one + eps), one - eps)
    vn = jnp.minimum(jnp.maximum(v * 2.0 - 1.0, -one + eps), one - eps)
    ix = ((un + 1.0) * RES - 1.0) * 0.5
    iy = ((vn + 1.0) * RES - 1.0) * 0.5
    x0 = ix.astype(jnp.int32)
    y0 = iy.astype(jnp.int32)
    wx1 = ix - x0.astype(jnp.float32)
    wy1 = iy - y0.astype(jnp.float32)
    wx0 = 1.0 - wx1
    wy0 = 1.0 - wy1
    f0 = y0 * RES + x0 + plane_base
    return f0, wy0 * wx0, wy0 * wx1, wy1 * wx0, wy1 * wx1


def _make_sc_call(cpw, n_pad):
    assert cpw % 2 == 0
    mesh = plsc.VectorSubcoreMesh(core_axis_name="c", subcore_axis_name="s")

    @functools.partial(
        pl.kernel,
        out_type=jax.ShapeDtypeStruct((n_pad * 3 * RANK,), jnp.float32),
        mesh=mesh,
        scratch_types=[
            pltpu.VMEM((3, 2 * CP), jnp.float32),
            pltpu.VMEM((2, 3, CP), jnp.int32),
            pltpu.VMEM((2, 12, CP), jnp.float32),
            pltpu.VMEM((2, 3, CP, 4 * RANK), jnp.float32),
            pltpu.VMEM((2, CP * 3 * RANK), jnp.float32),
            pltpu.SemaphoreType.DMA,
            pltpu.SemaphoreType.DMA,
            pltpu.SemaphoreType.DMA,
            pltpu.SemaphoreType.DMA,
        ],
        compiler_params=pltpu.CompilerParams(
            needs_layout_passes=False, use_tc_tiling_on_sc=False),
    )
    def sc_call(x_h, y_h, z_h, t4, feat, cbuf, ibuf, wbuf, gbuf, fbuf,
                sg0, sg1, sf0, sf1):
        wid = lax.axis_index("s") * NC + lax.axis_index("c")
        lane = lax.iota(jnp.int32, 16)
        wbase = wid * cpw * CP
        semg = (sg0, sg1)
        semf = (sf0, sf1)

        def stage_coords(first_chunk):
            off = wbase + first_chunk * CP
            for r, src in enumerate((x_h, y_h, z_h)):
                pltpu.sync_copy(src.at[pl.ds(off, 2 * CP)], cbuf.at[r])

        def pass1(s, half):
            def body(b):
                off = half * CP + b * LANES
                doff = b * LANES
                xv = _sanitize(cbuf[0, pl.ds(off, LANES)])
                yv = _sanitize(cbuf[1, pl.ds(off, LANES)])
                zv = _sanitize(cbuf[2, pl.ds(off, LANES)])
                for p, (u, v) in enumerate(((xv, yv), (xv, zv), (yv, zv))):
                    f0, wa, wb, wc, wd = _bilinear_uv(u, v, p * RES * RES)
                    ibuf[s, p, pl.ds(doff, LANES)] = f0
                    wbuf[s, 4 * p + 0, pl.ds(doff, LANES)] = wa
                    wbuf[s, 4 * p + 1, pl.ds(doff, LANES)] = wb
                    wbuf[s, 4 * p + 2, pl.ds(doff, LANES)] = wc
                    wbuf[s, 4 * p + 3, pl.ds(doff, LANES)] = wd

            plsc.parallel_loop(0, NB, 1)(body)

        def fire_gathers(s):
            return [pltpu.async_copy(t4.at[ibuf.at[s, k]], gbuf.at[s, k],
                                     semg[s])
                    for k in range(3)]

        def pass2(s):
            def body(b):
                off = b * LANES
                w = [wbuf[s, j, pl.ds(off, LANES)] for j in range(12)]
                for j in range(LANES):
                    n = off + j
                    for p in range(3):
                        wa = w[4 * p + 0][j]
                        wb = w[4 * p + 1][j]
                        wc = w[4 * p + 2][j]
                        wd = w[4 * p + 3][j]
                        r0a = gbuf[s, p, n, pl.ds(0, RANK)]
                        r0b = gbuf[s, p, n, pl.ds(RANK, RANK)]
                        r1a = gbuf[s, p, n, pl.ds(2 * RANK, RANK)]
                        r1b = gbuf[s, p, n, pl.ds(3 * RANK, RANK)]
                        acc = (wa * r0a + wb * r0b) + (wc * r1a + wd * r1b)
                        fbuf[s, pl.ds(n * 3 * RANK + p * RANK, RANK)] = acc

            plsc.parallel_loop(0, NB, 1)(body)

        def drain_featw(s):
            pltpu.make_async_copy(feat.at[pl.ds(0, CP * 3 * RANK)],
                                  fbuf.at[s], semf[s]).wait()

        def fire_featw(s, ci):
            pltpu.async_copy(
                fbuf.at[s],
                feat.at[pl.ds((wbase + ci * CP) * 3 * RANK, CP * 3 * RANK)],
                semf[s])

        stage_coords(0)

        def pair(k, carry):
            c0 = 2 * k
            pass1(0, 0)
            g0 = fire_gathers(0)
            pass1(1, 1)
            g1 = fire_gathers(1)

            @pl.when(k < cpw // 2 - 1)
            def _():
                stage_coords(c0 + 2)

            for c in g0:
                c.wait()

            @pl.when(k >= 1)
            def _():
                drain_featw(0)

            pass2(0)
            fire_featw(0, c0)
            for c in g1:
                c.wait()

            @pl.when(k >= 1)
            def _():
                drain_featw(1)

            pass2(1)
            fire_featw(1, c0 + 1)
            return carry

        lax.fori_loop(0, cpw // 2, pair, 0)
        drain_featw(0)
        drain_featw(1)

    return sc_call


def kernel(xyz, xy, xz, yz, W_mlp):
    n = xyz.shape[0]
    chunks = -(-n // CP)
    cpw = -(-chunks // NW)
    cpw += cpw % 2
    n_pad = cpw * NW * CP

    planes = jnp.concatenate([xy, xz, yz], axis=0)
    lpf = pl.pallas_call(
        _lpf_body,
        grid=(planes.shape[0] // 8,),
        in_specs=[pl.BlockSpec((8, RES, RES), lambda i: (i, 0, 0))],
        out_specs=pl.BlockSpec((8, RES, RES), lambda i: (i, 0, 0)),
        out_shape=jax.ShapeDtypeStruct(planes.shape, jnp.float32),
    )(planes)

    p_t = lpf.reshape(3, RANK, RES, RES).transpose(0, 2, 3, 1)
    p_sx = jnp.concatenate([p_t[:, :, 1:, :], p_t[:, :, -1:, :]], axis=2)
    q = jnp.concatenate([p_t, p_sx], axis=-1)
    q_sy = jnp.concatenate([q[:, 1:], q[:, -1:]], axis=1)
    t4 = jnp.concatenate([q, q_sy], axis=-1).reshape(3 * RES * RES, 4 * RANK)

    pads = ((0, n_pad - n),)
    xs = jnp.pad(xyz[:, 0], pads, constant_values=0.5)
    ys = jnp.pad(xyz[:, 1], pads, constant_values=0.5)
    zs = jnp.pad(xyz[:, 2], pads, constant_values=0.5)
    feat = _make_sc_call(cpw, n_pad)(xs, ys, zs, t4)

    bn = 4096
    f128 = feat.reshape(n_pad * 3 * RANK // 128, 128)
    wbig = jnp.einsum('ab,mk->akbm', jnp.eye(8, dtype=jnp.float32),
                      W_mlp).reshape(8 * 3 * RANK, 8 * OUT_D)
    out8 = pl.pallas_call(
        _mm_body,
        grid=(-(-n // bn),),
        in_specs=[pl.BlockSpec((bn * 3 * RANK // 128, 128), lambda i: (i, 0)),
                  pl.BlockSpec((8 * 3 * RANK, 8 * OUT_D), lambda i: (0, 0))],
        out_specs=pl.BlockSpec((bn // 8, 8 * OUT_D), lambda i: (i, 0)),
        out_shape=jax.ShapeDtypeStruct((n_pad // 8, 8 * OUT_D), jnp.float32),
    )(f128, wbig)
    return out8.reshape(n_pad, OUT_D)[:n]
